# baseline (speedup 1.0000x reference)
import functools

import jax
import jax.numpy as jnp
from jax import lax
from jax.experimental import pallas as pl
from jax.experimental.pallas import tpu as pltpu
from jax.experimental.pallas import tpu_sc as plsc

VOCAB = 32000
DIM = 4096
B = 4 * 2048

NUM_CORES = 2
NUM_SUBCORES = 16
NW = NUM_CORES * NUM_SUBCORES

B_SC = 5120
B_TC = B - B_SC
CHUNK = 8
NBUF = 3
TC_ROWS = 8


def _sc_body(table_hbm, idx_hbm, out_hbm, idx_v, rows, gsems, ssems):
    b_per_w = B_SC // NW
    nchunk = b_per_w // CHUNK
    wid = lax.axis_index("s") * NUM_CORES + lax.axis_index("c")
    base = wid * b_per_w

    pltpu.sync_copy(idx_hbm.at[pl.ds(base, b_per_w)], idx_v)

    def gather(g, s):
        pltpu.async_copy(
            table_hbm.at[idx_v.at[pl.ds(g * CHUNK, CHUNK)]],
            rows[s], gsems[s])

    def put(g, s):
        pltpu.async_copy(
            rows[s], out_hbm.at[pl.ds(base + g * CHUNK, CHUNK)], ssems[s])

    def wait_gather(s):
        pltpu.make_async_copy(
            table_hbm.at[idx_v.at[pl.ds(0, CHUNK)]], rows[s], gsems[s]).wait()

    def wait_put(s):
        pltpu.make_async_copy(
            rows[s], out_hbm.at[pl.ds(base, CHUNK)], ssems[s]).wait()

    for g in range(min(NBUF, nchunk)):
        gather(g, g % NBUF)

    for g in range(nchunk):
        s = g % NBUF
        wait_gather(s)
        put(g, s)
        gn = g + 2
        if NBUF <= gn < nchunk:
            sn = gn % NBUF
            wait_put(sn)
            gather(gn, sn)

    for g in range(max(0, nchunk - NBUF), nchunk):
        wait_put(g % NBUF)


def _sc_gather(weight, idx_sc):
    mesh = plsc.VectorSubcoreMesh(
        core_axis_name="c", subcore_axis_name="s",
        num_cores=NUM_CORES, num_subcores=NUM_SUBCORES,
    )
    return pl.kernel(
        _sc_body,
        out_type=jax.ShapeDtypeStruct((B_SC, DIM), jnp.float32),
        mesh=mesh,
        scratch_types=[
            pltpu.VMEM((B_SC // NW,), jnp.int32),
            [pltpu.VMEM((CHUNK, DIM), jnp.float32) for _ in range(NBUF)],
            [pltpu.SemaphoreType.DMA for _ in range(NBUF)],
            [pltpu.SemaphoreType.DMA for _ in range(NBUF)],
        ],
    )(weight, idx_sc)


def _tc_body(idx_ref, *refs):
    outs = refs[TC_ROWS]
    for j in range(TC_ROWS):
        outs[pl.ds(j, 1), :] = refs[j][0]


def _tc_gather(weight, idx_tc):
    n_steps = B_TC // TC_ROWS
    w3 = weight.reshape(VOCAB, 1, DIM)

    def in_map(j):
        return lambda i, idx_ref: (idx_ref[i * TC_ROWS + j], 0, 0)

    return pl.pallas_call(
        _tc_body,
        grid_spec=pltpu.PrefetchScalarGridSpec(
            num_scalar_prefetch=1,
            grid=(n_steps,),
            in_specs=[pl.BlockSpec((1, 1, DIM), in_map(j))
                      for j in range(TC_ROWS)],
            out_specs=pl.BlockSpec((TC_ROWS, DIM), lambda i, idx_ref: (i, 0)),
        ),
        out_shape=jax.ShapeDtypeStruct((B_TC, DIM), jnp.float32),
    )(idx_tc, *([w3] * TC_ROWS))


@jax.jit
def _embedding_lookup(weight, idx):
    out_sc = _sc_gather(weight, idx[:B_SC])
    out_tc = _tc_gather(weight, idx[B_SC:])
    return jnp.concatenate([out_sc, out_tc], axis=0)


def kernel(x, weight):
    idx = x.reshape(-1).astype(jnp.int32)
    out = _embedding_lookup(weight, idx)
    return out.reshape(x.shape + (DIM,))


# speedup vs baseline: 7.9322x; 7.9322x over previous
import jax
import jax.numpy as jnp
from jax import lax
from jax.experimental import pallas as pl
from jax.experimental.pallas import tpu as pltpu
from jax.experimental.pallas import tpu_sc as plsc

VOCAB = 32000
DIM = 4096
B = 4 * 2048

NUM_CORES = 2
NUM_SUBCORES = 16
NW = NUM_CORES * NUM_SUBCORES
B_PER_W = B // NW
CHUNK = 4
NBUF = 7
NCHUNK = B_PER_W // CHUNK
LEAD = 4


def _emb_body(table_hbm, idx_hbm, out_hbm, idx_v, rows, gsems, ssems):
    wid = lax.axis_index("s") * NUM_CORES + lax.axis_index("c")
    base = wid * B_PER_W

    pltpu.sync_copy(idx_hbm.at[pl.ds(wid * NCHUNK, NCHUNK)], idx_v)

    def gather(g, s):
        pltpu.async_copy(
            table_hbm.at[idx_v.at[g]],
            rows[s], gsems[s])

    def put(g, s):
        pltpu.async_copy(
            rows[s], out_hbm.at[pl.ds(base + g * CHUNK, CHUNK)], ssems[s])

    def wait_gather(s):
        pltpu.make_async_copy(
            table_hbm.at[idx_v.at[0]], rows[s], gsems[s]).wait()

    def wait_put(s):
        pltpu.make_async_copy(
            rows[s], out_hbm.at[pl.ds(base, CHUNK)], ssems[s]).wait()

    for g in range(NBUF):
        gather(g, g % NBUF)

    for g in range(NCHUNK):
        s = g % NBUF
        wait_gather(s)
        put(g, s)
        gn = g + LEAD
        if NBUF <= gn < NCHUNK:
            sn = gn % NBUF
            wait_put(sn)
            gather(gn, sn)

    for g in range(NCHUNK - NBUF, NCHUNK):
        wait_put(g % NBUF)


@jax.jit
def _embedding_lookup(weight, idx):
    mesh = plsc.VectorSubcoreMesh(
        core_axis_name="c", subcore_axis_name="s",
        num_cores=NUM_CORES, num_subcores=NUM_SUBCORES,
    )
    return pl.kernel(
        _emb_body,
        out_type=jax.ShapeDtypeStruct((B, DIM), jnp.float32),
        mesh=mesh,
        scratch_types=[
            pltpu.VMEM((NCHUNK, CHUNK), jnp.int32),
            [pltpu.VMEM((CHUNK, DIM), jnp.float32) for _ in range(NBUF)],
            [pltpu.SemaphoreType.DMA for _ in range(NBUF)],
            [pltpu.SemaphoreType.DMA for _ in range(NBUF)],
        ],
    )(weight, idx.reshape(-1, CHUNK))


def kernel(x, weight):
    idx = x.reshape(-1).astype(jnp.int32)
    out = _embedding_lookup(weight, idx)
    return out.reshape(x.shape + (DIM,))


# speedup vs baseline: 7.9561x; 1.0030x over previous
import jax
import jax.numpy as jnp
from jax import lax
from jax.experimental import pallas as pl
from jax.experimental.pallas import tpu as pltpu
from jax.experimental.pallas import tpu_sc as plsc

VOCAB = 32000
DIM = 4096
XROWS = 4
XCOLS = 2048
B = XROWS * XCOLS

NUM_CORES = 2
NUM_SUBCORES = 16
NW = NUM_CORES * NUM_SUBCORES
B_PER_W = B // NW
W_PER_XROW = XCOLS // B_PER_W
CHUNK = 8
NBUF = 3
NCHUNK = B_PER_W // CHUNK


def _emb_body(table_hbm, x_hbm, out_hbm, idx_v, rows, gsems, ssems):
    wid = lax.axis_index("s") * NUM_CORES + lax.axis_index("c")
    xr = wid // W_PER_XROW
    c0 = (wid % W_PER_XROW) * B_PER_W

    pltpu.sync_copy(x_hbm.at[xr, pl.ds(c0, B_PER_W)], idx_v)

    def gather(g, s):
        pltpu.async_copy(
            table_hbm.at[idx_v.at[pl.ds(g * CHUNK, CHUNK)]],
            rows[s], gsems[s])

    def put(g, s):
        pltpu.async_copy(
            rows[s], out_hbm.at[xr, pl.ds(c0 + g * CHUNK, CHUNK)], ssems[s])

    def wait_gather(s):
        pltpu.make_async_copy(
            table_hbm.at[idx_v.at[pl.ds(0, CHUNK)]], rows[s], gsems[s]).wait()

    def wait_put(s):
        pltpu.make_async_copy(
            rows[s], out_hbm.at[xr, pl.ds(c0, CHUNK)], ssems[s]).wait()

    for g in range(NBUF):
        gather(g, g % NBUF)

    for g in range(NCHUNK):
        s = g % NBUF
        wait_gather(s)
        put(g, s)
        gn = g + 2
        if NBUF <= gn < NCHUNK:
            sn = gn % NBUF
            wait_put(sn)
            gather(gn, sn)

    for g in range(NCHUNK - NBUF, NCHUNK):
        wait_put(g % NBUF)


@jax.jit
def _embedding_lookup(weight, x):
    mesh = plsc.VectorSubcoreMesh(
        core_axis_name="c", subcore_axis_name="s",
        num_cores=NUM_CORES, num_subcores=NUM_SUBCORES,
    )
    return pl.kernel(
        _emb_body,
        out_type=jax.ShapeDtypeStruct((XROWS, XCOLS, DIM), jnp.float32),
        mesh=mesh,
        scratch_types=[
            pltpu.VMEM((B_PER_W,), jnp.int32),
            [pltpu.VMEM((CHUNK, DIM), jnp.float32) for _ in range(NBUF)],
            [pltpu.SemaphoreType.DMA for _ in range(NBUF)],
            [pltpu.SemaphoreType.DMA for _ in range(NBUF)],
        ],
    )(weight, x)


def kernel(x, weight):
    return _embedding_lookup(weight, x)


# speedup vs baseline: 7.9677x; 1.0015x over previous
import jax
import jax.numpy as jnp
from jax import lax
from jax.experimental import pallas as pl
from jax.experimental.pallas import tpu as pltpu
from jax.experimental.pallas import tpu_sc as plsc

VOCAB = 32000
DIM = 4096
XROWS = 4
XCOLS = 2048
B = XROWS * XCOLS

NUM_CORES = 2
NUM_SUBCORES = 16
NW = NUM_CORES * NUM_SUBCORES
B_PER_W = B // NW
W_PER_XROW = XCOLS // B_PER_W
CHUNK = 8
NBUF = 3
NCHUNK = B_PER_W // CHUNK


def _emb_body(table_hbm, x_hbm, out_hbm, idx_v, rows, gsems, ssems):
    wid = lax.axis_index("s") * NUM_CORES + lax.axis_index("c")
    xr = wid // W_PER_XROW
    c0 = (wid % W_PER_XROW) * B_PER_W

    pltpu.sync_copy(x_hbm.at[xr, pl.ds(c0, B_PER_W)], idx_v)

    def gather(g, s):
        pltpu.async_copy(
            table_hbm.at[idx_v.at[pl.ds(g * CHUNK, CHUNK)]],
            rows[s], gsems[s], priority=1)

    def put(g, s):
        pltpu.async_copy(
            rows[s], out_hbm.at[xr, pl.ds(c0 + g * CHUNK, CHUNK)], ssems[s])

    def wait_gather(s):
        pltpu.make_async_copy(
            table_hbm.at[idx_v.at[pl.ds(0, CHUNK)]], rows[s], gsems[s]).wait()

    def wait_put(s):
        pltpu.make_async_copy(
            rows[s], out_hbm.at[xr, pl.ds(c0, CHUNK)], ssems[s]).wait()

    for g in range(NBUF):
        gather(g, g % NBUF)

    for g in range(NCHUNK):
        s = g % NBUF
        wait_gather(s)
        put(g, s)
        gn = g + 2
        if NBUF <= gn < NCHUNK:
            sn = gn % NBUF
            wait_put(sn)
            gather(gn, sn)

    for g in range(NCHUNK - NBUF, NCHUNK):
        wait_put(g % NBUF)


@jax.jit
def _embedding_lookup(weight, x):
    mesh = plsc.VectorSubcoreMesh(
        core_axis_name="c", subcore_axis_name="s",
        num_cores=NUM_CORES, num_subcores=NUM_SUBCORES,
    )
    return pl.kernel(
        _emb_body,
        out_type=jax.ShapeDtypeStruct((XROWS, XCOLS, DIM), jnp.float32),
        mesh=mesh,
        scratch_types=[
            pltpu.VMEM((B_PER_W,), jnp.int32),
            [pltpu.VMEM((CHUNK, DIM), jnp.float32) for _ in range(NBUF)],
            [pltpu.SemaphoreType.DMA for _ in range(NBUF)],
            [pltpu.SemaphoreType.DMA for _ in range(NBUF)],
        ],
    )(weight, x)


def kernel(x, weight):
    return _embedding_lookup(weight, x)
